# baseline (speedup 1.0000x reference)
import functools

import jax
import jax.numpy as jnp
from jax import lax
from jax.experimental import pallas as pl
from jax.experimental.pallas import tpu as pltpu
from jax.experimental.pallas import tpu_sc as plsc

D_MODEL = 2048
NUM_EXPERTS = 16
TOP_K = 2
TOKENS = 16384

SC_TOKENS = 2048
TC_TOKENS = TOKENS - SC_TOKENS
BLOCK_T = 2048

N_WORKERS = 32
Q_PER_W = SC_TOKENS // N_WORKERS
CHUNK = 16
LANES = 16
N_CHUNKS = Q_PER_W // CHUNK


def _tc_body(hs_ref, w_ref, rw_ref, idx_ref):
    logits = lax.dot_general(
        hs_ref[...], w_ref[...],
        dimension_numbers=(((1,), (1,)), ((), ())),
        preferred_element_type=jnp.float32,
    )

    iota = lax.broadcasted_iota(jnp.int32, logits.shape, 1)
    m1 = jnp.max(logits, axis=-1, keepdims=True)
    i1 = jnp.min(jnp.where(logits == m1, iota, NUM_EXPERTS),
                 axis=-1, keepdims=True)
    masked = jnp.where(iota == i1, -jnp.inf, logits)
    m2 = jnp.max(masked, axis=-1, keepdims=True)
    i2 = jnp.min(jnp.where(masked == m2, iota, NUM_EXPERTS),
                 axis=-1, keepdims=True)

    e2 = jnp.exp(m2 - m1)
    denom = 1.0 + e2
    rw_ref[...] = jnp.concatenate([1.0 / denom, e2 / denom], axis=-1)
    idx_ref[...] = jnp.concatenate([i1, i2], axis=-1)


def _tc_router(hidden_states, W):
    n_blocks = TC_TOKENS // BLOCK_T
    return pl.pallas_call(
        _tc_body,
        grid=(n_blocks,),
        in_specs=[
            pl.BlockSpec((BLOCK_T, D_MODEL), lambda i: (i, 0)),
            pl.BlockSpec((NUM_EXPERTS, D_MODEL), lambda i: (0, 0)),
        ],
        out_specs=[
            pl.BlockSpec((BLOCK_T, TOP_K), lambda i: (i, 0)),
            pl.BlockSpec((BLOCK_T, TOP_K), lambda i: (i, 0)),
        ],
        out_shape=[
            jax.ShapeDtypeStruct((TC_TOKENS, TOP_K), jnp.float32),
            jax.ShapeDtypeStruct((TC_TOKENS, TOP_K), jnp.int32),
        ],
    )(hidden_states, W)


_GDN = lax.GatherDimensionNumbers(
    offset_dims=(), collapsed_slice_dims=(0,), start_index_map=(0,))


def _rot(v, k):
    iota = lax.iota(jnp.int32, LANES)
    idx = jnp.bitwise_and(iota + k, LANES - 1)
    return lax.gather(v, idx[:, None], _GDN, (1,),
                      mode=lax.GatherScatterMode.PROMISE_IN_BOUNDS)


def _tree(v, op):
    for k in (8, 4, 2, 1):
        v = op(v, _rot(v, k))
    return v


def _round_bf16(v):
    u = lax.bitcast_convert_type(v, jnp.int32)
    r = (u + 0x7FFF + jnp.bitwise_and(lax.shift_right_logical(u, 16), 1))
    r = jnp.bitwise_and(r, jnp.int32(-65536))
    return lax.bitcast_convert_type(r, jnp.float32)


def _lane_select(vecs):
    iota = lax.iota(jnp.int32, LANES)
    out = vecs[0]
    for e in range(1, LANES):
        out = jnp.where(iota == e, vecs[e], out)
    return out


def _sc_router(hidden_states, W):
    mesh = plsc.VectorSubcoreMesh(core_axis_name="c", subcore_axis_name="s")

    @functools.partial(
        pl.kernel,
        mesh=mesh,
        out_type=[
            jax.ShapeDtypeStruct((SC_TOKENS, LANES), jnp.float32),
            jax.ShapeDtypeStruct((SC_TOKENS, LANES), jnp.float32),
            jax.ShapeDtypeStruct((SC_TOKENS, LANES), jnp.int32),
            jax.ShapeDtypeStruct((SC_TOKENS, LANES), jnp.int32),
        ],
        scratch_types=[
            pltpu.VMEM((NUM_EXPERTS, D_MODEL), jnp.float32),
            pltpu.VMEM((CHUNK, D_MODEL), jnp.float32),
            pltpu.VMEM((CHUNK, LANES), jnp.float32),
            pltpu.VMEM((CHUNK, LANES), jnp.float32),
            pltpu.VMEM((CHUNK, LANES), jnp.int32),
            pltpu.VMEM((CHUNK, LANES), jnp.int32),
        ],
    )
    def sc_kernel(hs_hbm, w_hbm, w1_hbm, w2_hbm, i1_hbm, i2_hbm,
                  w_v, rows_v, w1r_v, w2r_v, i1r_v, i2r_v):
        wid = lax.axis_index("s") * 2 + lax.axis_index("c")
        base = TC_TOKENS + wid * Q_PER_W

        pltpu.sync_copy(w_hbm, w_v)

        @pl.loop(0, N_CHUNKS)
        def _chunk(ci):
            pltpu.sync_copy(hs_hbm.at[pl.ds(base + ci * CHUNK, CHUNK)],
                            rows_v)

            for t in range(0, CHUNK, 2):
                zero = jnp.zeros((LANES,), jnp.float32)
                init = (zero,) * (2 * NUM_EXPERTS)

                def jbody(j, accs, _t=t):
                    d0 = j * LANES
                    h0 = _round_bf16(rows_v[_t, pl.ds(d0, LANES)])
                    h1 = _round_bf16(rows_v[_t + 1, pl.ds(d0, LANES)])
                    a = []
                    b = []
                    for e in range(NUM_EXPERTS):
                        we = w_v[e, pl.ds(d0, LANES)]
                        a.append(accs[e] + h0 * we)
                        b.append(accs[NUM_EXPERTS + e] + h1 * we)
                    return tuple(a) + tuple(b)

                accs = lax.fori_loop(0, D_MODEL // LANES, jbody, init)

                for half in range(2):
                    red = [_tree(accs[half * NUM_EXPERTS + e], jnp.add)
                           for e in range(NUM_EXPERTS)]
                    m1 = red[0]
                    i1 = jnp.zeros((LANES,), jnp.int32)
                    m2 = jnp.full((LANES,), -jnp.inf, jnp.float32)
                    i2 = jnp.full((LANES,), NUM_EXPERTS, jnp.int32)
                    for e in range(1, NUM_EXPERTS):
                        ev = jnp.full((LANES,), e, jnp.int32)
                        gt1 = red[e] > m1
                        gt2 = red[e] > m2
                        m2 = jnp.where(gt1, m1, jnp.where(gt2, red[e], m2))
                        i2 = jnp.where(gt1, i1, jnp.where(gt2, ev, i2))
                        m1 = jnp.where(gt1, red[e], m1)
                        i1 = jnp.where(gt1, ev, i1)
                    e2v = jnp.exp(m2 - m1)
                    w1v = 1.0 / (1.0 + e2v)
                    w2v = e2v * w1v
                    w1r_v[t + half] = w1v
                    w2r_v[t + half] = w2v
                    i1r_v[t + half] = i1
                    i2r_v[t + half] = i2

            row0 = wid * Q_PER_W + ci * CHUNK
            pltpu.sync_copy(w1r_v, w1_hbm.at[pl.ds(row0, CHUNK)])
            pltpu.sync_copy(w2r_v, w2_hbm.at[pl.ds(row0, CHUNK)])
            pltpu.sync_copy(i1r_v, i1_hbm.at[pl.ds(row0, CHUNK)])
            pltpu.sync_copy(i2r_v, i2_hbm.at[pl.ds(row0, CHUNK)])

    u = lax.bitcast_convert_type(W, jnp.int32)
    r = u + 0x7FFF + jnp.bitwise_and(lax.shift_right_logical(u, 16), 1)
    w_r = lax.bitcast_convert_type(
        jnp.bitwise_and(r, jnp.int32(-65536)), jnp.float32)
    return sc_kernel(hidden_states, w_r)


def kernel(hidden_states, W):
    rw_tc, idx_tc = _tc_router(hidden_states, W)
    w1, w2, i1, i2 = _sc_router(hidden_states, W)
    rw_sc = jnp.stack([w1[:, 0], w2[:, 0]], axis=-1)
    idx_sc = jnp.stack([i1[:, 0], i2[:, 0]], axis=-1)
    rw = jnp.concatenate([rw_tc, rw_sc], axis=0)
    idx = jnp.concatenate([idx_tc, idx_sc], axis=0)
    return (rw, idx)


# speedup vs baseline: 1.1045x; 1.1045x over previous
import functools

import jax
import jax.numpy as jnp
from jax import lax
from jax.experimental import pallas as pl
from jax.experimental.pallas import tpu as pltpu
from jax.experimental.pallas import tpu_sc as plsc

D_MODEL = 2048
NUM_EXPERTS = 16
TOP_K = 2
TOKENS = 16384

SC_TOKENS = 1024
TC_TOKENS = TOKENS - SC_TOKENS
CK = 1024
NBUF = 3

N_WORKERS = 32
Q_PER_W = SC_TOKENS // N_WORKERS
CHUNK = 16
LANES = 16
N_CHUNKS = Q_PER_W // CHUNK


def _tc_top2(logits):
    iota = lax.broadcasted_iota(jnp.int32, logits.shape, 1)
    m1 = jnp.max(logits, axis=-1, keepdims=True)
    i1 = jnp.min(jnp.where(logits == m1, iota, NUM_EXPERTS),
                 axis=-1, keepdims=True)
    masked = jnp.where(iota == i1, -jnp.inf, logits)
    m2 = jnp.max(masked, axis=-1, keepdims=True)
    i2 = jnp.min(jnp.where(masked == m2, iota, NUM_EXPERTS),
                 axis=-1, keepdims=True)
    e2 = jnp.exp(m2 - m1)
    denom = 1.0 + e2
    rw = jnp.concatenate([1.0 / denom, e2 / denom], axis=-1)
    idx = jnp.concatenate([i1, i2], axis=-1)
    return rw, idx


def _tc_body(hs_hbm, w_ref, rw_ref, idx_ref, buf, sems):
    n_chunks = TC_TOKENS // CK

    def _copy(i, b):
        return pltpu.make_async_copy(
            hs_hbm.at[pl.ds(i * CK, CK)], buf.at[b], sems.at[b])

    for b in range(NBUF):
        _copy(b, b).start()

    w = w_ref[...]
    dn = (((1,), (1,)), ((), ()))

    def round_body(r, _):
        for b in range(NBUF):
            i = r * NBUF + b
            _copy(i, b).wait()
            logits = lax.dot_general(buf[b], w, dn,
                                     preferred_element_type=jnp.float32)
            rw, idx = _tc_top2(logits)
            rw_ref[pl.ds(i * CK, CK), :] = rw
            idx_ref[pl.ds(i * CK, CK), :] = idx
            nxt = i + NBUF

            @pl.when(nxt < n_chunks)
            def _():
                _copy(nxt, b).start()
        return 0

    lax.fori_loop(0, n_chunks // NBUF, round_body, 0)


def _tc_router(hidden_states, W):
    return pl.pallas_call(
        _tc_body,
        in_specs=[
            pl.BlockSpec(memory_space=pltpu.MemorySpace.HBM),
            pl.BlockSpec((NUM_EXPERTS, D_MODEL), lambda: (0, 0)),
        ],
        out_specs=[
            pl.BlockSpec((TC_TOKENS, TOP_K), lambda: (0, 0)),
            pl.BlockSpec((TC_TOKENS, TOP_K), lambda: (0, 0)),
        ],
        out_shape=[
            jax.ShapeDtypeStruct((TC_TOKENS, TOP_K), jnp.float32),
            jax.ShapeDtypeStruct((TC_TOKENS, TOP_K), jnp.int32),
        ],
        scratch_shapes=[
            pltpu.VMEM((NBUF, CK, D_MODEL), jnp.float32),
            pltpu.SemaphoreType.DMA((NBUF,)),
        ],
    )(hidden_states, W)


_GDN = lax.GatherDimensionNumbers(
    offset_dims=(), collapsed_slice_dims=(0,), start_index_map=(0,))


def _rot(v, k):
    iota = lax.iota(jnp.int32, LANES)
    idx = jnp.bitwise_and(iota + k, LANES - 1)
    return lax.gather(v, idx[:, None], _GDN, (1,),
                      mode=lax.GatherScatterMode.PROMISE_IN_BOUNDS)


def _tree(v, op):
    for k in (8, 4, 2, 1):
        v = op(v, _rot(v, k))
    return v


def _round_bf16(v):
    u = lax.bitcast_convert_type(v, jnp.int32)
    r = (u + 0x7FFF + jnp.bitwise_and(lax.shift_right_logical(u, 16), 1))
    r = jnp.bitwise_and(r, jnp.int32(-65536))
    return lax.bitcast_convert_type(r, jnp.float32)


def _lane_select(vecs):
    iota = lax.iota(jnp.int32, LANES)
    out = vecs[0]
    for e in range(1, LANES):
        out = jnp.where(iota == e, vecs[e], out)
    return out


def _sc_router(hidden_states, W):
    mesh = plsc.VectorSubcoreMesh(core_axis_name="c", subcore_axis_name="s")

    @functools.partial(
        pl.kernel,
        mesh=mesh,
        out_type=[
            jax.ShapeDtypeStruct((SC_TOKENS, LANES), jnp.float32),
            jax.ShapeDtypeStruct((SC_TOKENS, LANES), jnp.float32),
            jax.ShapeDtypeStruct((SC_TOKENS, LANES), jnp.int32),
            jax.ShapeDtypeStruct((SC_TOKENS, LANES), jnp.int32),
        ],
        scratch_types=[
            pltpu.VMEM((NUM_EXPERTS, D_MODEL), jnp.float32),
            pltpu.VMEM((CHUNK, D_MODEL), jnp.float32),
            pltpu.VMEM((CHUNK, LANES), jnp.float32),
            pltpu.VMEM((CHUNK, LANES), jnp.float32),
            pltpu.VMEM((CHUNK, LANES), jnp.int32),
            pltpu.VMEM((CHUNK, LANES), jnp.int32),
        ],
    )
    def sc_kernel(hs_hbm, w_hbm, w1_hbm, w2_hbm, i1_hbm, i2_hbm,
                  w_v, rows_v, w1r_v, w2r_v, i1r_v, i2r_v):
        wid = lax.axis_index("s") * 2 + lax.axis_index("c")
        base = TC_TOKENS + wid * Q_PER_W

        pltpu.sync_copy(w_hbm, w_v)

        @pl.loop(0, N_CHUNKS)
        def _chunk(ci):
            pltpu.sync_copy(hs_hbm.at[pl.ds(base + ci * CHUNK, CHUNK)],
                            rows_v)

            for t in range(0, CHUNK, 2):
                zero = jnp.zeros((LANES,), jnp.float32)
                init = (zero,) * (2 * NUM_EXPERTS)

                def jbody(j, accs, _t=t):
                    d0 = j * LANES
                    h0 = _round_bf16(rows_v[_t, pl.ds(d0, LANES)])
                    h1 = _round_bf16(rows_v[_t + 1, pl.ds(d0, LANES)])
                    a = []
                    b = []
                    for e in range(NUM_EXPERTS):
                        we = w_v[e, pl.ds(d0, LANES)]
                        a.append(accs[e] + h0 * we)
                        b.append(accs[NUM_EXPERTS + e] + h1 * we)
                    return tuple(a) + tuple(b)

                accs = lax.fori_loop(0, D_MODEL // LANES, jbody, init,
                                     unroll=4)

                for half in range(2):
                    red = [_tree(accs[half * NUM_EXPERTS + e], jnp.add)
                           for e in range(NUM_EXPERTS)]
                    m1 = red[0]
                    i1 = jnp.zeros((LANES,), jnp.int32)
                    m2 = jnp.full((LANES,), -jnp.inf, jnp.float32)
                    i2 = jnp.full((LANES,), NUM_EXPERTS, jnp.int32)
                    for e in range(1, NUM_EXPERTS):
                        ev = jnp.full((LANES,), e, jnp.int32)
                        gt1 = red[e] > m1
                        gt2 = red[e] > m2
                        m2 = jnp.where(gt1, m1, jnp.where(gt2, red[e], m2))
                        i2 = jnp.where(gt1, i1, jnp.where(gt2, ev, i2))
                        m1 = jnp.where(gt1, red[e], m1)
                        i1 = jnp.where(gt1, ev, i1)
                    e2v = jnp.exp(m2 - m1)
                    w1v = 1.0 / (1.0 + e2v)
                    w2v = e2v * w1v
                    w1r_v[t + half] = w1v
                    w2r_v[t + half] = w2v
                    i1r_v[t + half] = i1
                    i2r_v[t + half] = i2

            row0 = wid * Q_PER_W + ci * CHUNK
            pltpu.sync_copy(w1r_v, w1_hbm.at[pl.ds(row0, CHUNK)])
            pltpu.sync_copy(w2r_v, w2_hbm.at[pl.ds(row0, CHUNK)])
            pltpu.sync_copy(i1r_v, i1_hbm.at[pl.ds(row0, CHUNK)])
            pltpu.sync_copy(i2r_v, i2_hbm.at[pl.ds(row0, CHUNK)])

    u = lax.bitcast_convert_type(W, jnp.int32)
    r = u + 0x7FFF + jnp.bitwise_and(lax.shift_right_logical(u, 16), 1)
    w_r = lax.bitcast_convert_type(
        jnp.bitwise_and(r, jnp.int32(-65536)), jnp.float32)
    return sc_kernel(hidden_states, w_r)


def kernel(hidden_states, W):
    rw_tc, idx_tc = _tc_router(hidden_states, W)
    w1, w2, i1, i2 = _sc_router(hidden_states, W)
    rw_sc = jnp.stack([w1[:, 0], w2[:, 0]], axis=-1)
    idx_sc = jnp.stack([i1[:, 0], i2[:, 0]], axis=-1)
    rw = jnp.concatenate([rw_tc, rw_sc], axis=0)
    idx = jnp.concatenate([idx_tc, idx_sc], axis=0)
    return (rw, idx)


# speedup vs baseline: 1.7355x; 1.5713x over previous
import functools

import jax
import jax.numpy as jnp
from jax import lax
from jax.experimental import pallas as pl
from jax.experimental.pallas import tpu as pltpu
from jax.experimental.pallas import tpu_sc as plsc

D_MODEL = 2048
NUM_EXPERTS = 16
TOP_K = 2
TOKENS = 16384

SC_TOKENS = 0
TC_TOKENS = TOKENS - SC_TOKENS
CK = 1024
NBUF = 4

N_WORKERS = 32
Q_PER_W = SC_TOKENS // N_WORKERS
CHUNK = 16
LANES = 16
N_CHUNKS = Q_PER_W // CHUNK


def _tc_top2(logits):
    iota = lax.broadcasted_iota(jnp.int32, logits.shape, 1)
    m1 = jnp.max(logits, axis=-1, keepdims=True)
    i1 = jnp.min(jnp.where(logits == m1, iota, NUM_EXPERTS),
                 axis=-1, keepdims=True)
    masked = jnp.where(iota == i1, -jnp.inf, logits)
    m2 = jnp.max(masked, axis=-1, keepdims=True)
    i2 = jnp.min(jnp.where(masked == m2, iota, NUM_EXPERTS),
                 axis=-1, keepdims=True)
    e2 = jnp.exp(m2 - m1)
    denom = 1.0 + e2
    rw = jnp.concatenate([1.0 / denom, e2 / denom], axis=-1)
    idx = jnp.concatenate([i1, i2], axis=-1)
    return rw, idx


def _tc_body(hs_hbm, w_ref, rw_ref, idx_ref, buf, sems):
    n_chunks = TC_TOKENS // CK

    def _copy(i, b):
        return pltpu.make_async_copy(
            hs_hbm.at[pl.ds(i * CK, CK)], buf.at[b], sems.at[b])

    for b in range(NBUF):
        _copy(b, b).start()

    w = w_ref[...]
    dn = (((1,), (1,)), ((), ()))

    def round_body(r, _):
        for b in range(NBUF):
            i = r * NBUF + b
            _copy(i, b).wait()
            logits = lax.dot_general(buf[b], w, dn,
                                     preferred_element_type=jnp.float32)
            rw, idx = _tc_top2(logits)
            rw_ref[pl.ds(i * CK, CK), :] = rw
            idx_ref[pl.ds(i * CK, CK), :] = idx
            nxt = i + NBUF

            @pl.when(nxt < n_chunks)
            def _():
                _copy(nxt, b).start()
        return 0

    lax.fori_loop(0, n_chunks // NBUF, round_body, 0)


def _tc_router(hidden_states, W):
    return pl.pallas_call(
        _tc_body,
        in_specs=[
            pl.BlockSpec(memory_space=pltpu.MemorySpace.HBM),
            pl.BlockSpec((NUM_EXPERTS, D_MODEL), lambda: (0, 0)),
        ],
        out_specs=[
            pl.BlockSpec((TC_TOKENS, TOP_K), lambda: (0, 0)),
            pl.BlockSpec((TC_TOKENS, TOP_K), lambda: (0, 0)),
        ],
        out_shape=[
            jax.ShapeDtypeStruct((TC_TOKENS, TOP_K), jnp.float32),
            jax.ShapeDtypeStruct((TC_TOKENS, TOP_K), jnp.int32),
        ],
        scratch_shapes=[
            pltpu.VMEM((NBUF, CK, D_MODEL), jnp.float32),
            pltpu.SemaphoreType.DMA((NBUF,)),
        ],
    )(hidden_states, W)


_GDN = lax.GatherDimensionNumbers(
    offset_dims=(), collapsed_slice_dims=(0,), start_index_map=(0,))


def _rot(v, k):
    iota = lax.iota(jnp.int32, LANES)
    idx = jnp.bitwise_and(iota + k, LANES - 1)
    return lax.gather(v, idx[:, None], _GDN, (1,),
                      mode=lax.GatherScatterMode.PROMISE_IN_BOUNDS)


def _tree(v, op):
    for k in (8, 4, 2, 1):
        v = op(v, _rot(v, k))
    return v


def _round_bf16(v):
    u = lax.bitcast_convert_type(v, jnp.int32)
    r = (u + 0x7FFF + jnp.bitwise_and(lax.shift_right_logical(u, 16), 1))
    r = jnp.bitwise_and(r, jnp.int32(-65536))
    return lax.bitcast_convert_type(r, jnp.float32)


def _lane_select(vecs):
    iota = lax.iota(jnp.int32, LANES)
    out = vecs[0]
    for e in range(1, LANES):
        out = jnp.where(iota == e, vecs[e], out)
    return out


def _sc_router(hidden_states, W):
    mesh = plsc.VectorSubcoreMesh(core_axis_name="c", subcore_axis_name="s")

    @functools.partial(
        pl.kernel,
        mesh=mesh,
        out_type=[
            jax.ShapeDtypeStruct((SC_TOKENS, LANES), jnp.float32),
            jax.ShapeDtypeStruct((SC_TOKENS, LANES), jnp.float32),
            jax.ShapeDtypeStruct((SC_TOKENS, LANES), jnp.int32),
            jax.ShapeDtypeStruct((SC_TOKENS, LANES), jnp.int32),
        ],
        scratch_types=[
            pltpu.VMEM((NUM_EXPERTS, D_MODEL), jnp.float32),
            pltpu.VMEM((CHUNK, D_MODEL), jnp.float32),
            pltpu.VMEM((CHUNK, LANES), jnp.float32),
            pltpu.VMEM((CHUNK, LANES), jnp.float32),
            pltpu.VMEM((CHUNK, LANES), jnp.int32),
            pltpu.VMEM((CHUNK, LANES), jnp.int32),
        ],
    )
    def sc_kernel(hs_hbm, w_hbm, w1_hbm, w2_hbm, i1_hbm, i2_hbm,
                  w_v, rows_v, w1r_v, w2r_v, i1r_v, i2r_v):
        wid = lax.axis_index("s") * 2 + lax.axis_index("c")
        base = TC_TOKENS + wid * Q_PER_W

        pltpu.sync_copy(w_hbm, w_v)

        @pl.loop(0, N_CHUNKS)
        def _chunk(ci):
            pltpu.sync_copy(hs_hbm.at[pl.ds(base + ci * CHUNK, CHUNK)],
                            rows_v)

            for t in range(0, CHUNK, 2):
                zero = jnp.zeros((LANES,), jnp.float32)
                init = (zero,) * (2 * NUM_EXPERTS)

                def jbody(j, accs, _t=t):
                    d0 = j * LANES
                    h0 = _round_bf16(rows_v[_t, pl.ds(d0, LANES)])
                    h1 = _round_bf16(rows_v[_t + 1, pl.ds(d0, LANES)])
                    a = []
                    b = []
                    for e in range(NUM_EXPERTS):
                        we = w_v[e, pl.ds(d0, LANES)]
                        a.append(accs[e] + h0 * we)
                        b.append(accs[NUM_EXPERTS + e] + h1 * we)
                    return tuple(a) + tuple(b)

                accs = lax.fori_loop(0, D_MODEL // LANES, jbody, init,
                                     unroll=4)

                for half in range(2):
                    red = [_tree(accs[half * NUM_EXPERTS + e], jnp.add)
                           for e in range(NUM_EXPERTS)]
                    m1 = red[0]
                    i1 = jnp.zeros((LANES,), jnp.int32)
                    m2 = jnp.full((LANES,), -jnp.inf, jnp.float32)
                    i2 = jnp.full((LANES,), NUM_EXPERTS, jnp.int32)
                    for e in range(1, NUM_EXPERTS):
                        ev = jnp.full((LANES,), e, jnp.int32)
                        gt1 = red[e] > m1
                        gt2 = red[e] > m2
                        m2 = jnp.where(gt1, m1, jnp.where(gt2, red[e], m2))
                        i2 = jnp.where(gt1, i1, jnp.where(gt2, ev, i2))
                        m1 = jnp.where(gt1, red[e], m1)
                        i1 = jnp.where(gt1, ev, i1)
                    e2v = jnp.exp(m2 - m1)
                    w1v = 1.0 / (1.0 + e2v)
                    w2v = e2v * w1v
                    w1r_v[t + half] = w1v
                    w2r_v[t + half] = w2v
                    i1r_v[t + half] = i1
                    i2r_v[t + half] = i2

            row0 = wid * Q_PER_W + ci * CHUNK
            pltpu.sync_copy(w1r_v, w1_hbm.at[pl.ds(row0, CHUNK)])
            pltpu.sync_copy(w2r_v, w2_hbm.at[pl.ds(row0, CHUNK)])
            pltpu.sync_copy(i1r_v, i1_hbm.at[pl.ds(row0, CHUNK)])
            pltpu.sync_copy(i2r_v, i2_hbm.at[pl.ds(row0, CHUNK)])

    u = lax.bitcast_convert_type(W, jnp.int32)
    r = u + 0x7FFF + jnp.bitwise_and(lax.shift_right_logical(u, 16), 1)
    w_r = lax.bitcast_convert_type(
        jnp.bitwise_and(r, jnp.int32(-65536)), jnp.float32)
    return sc_kernel(hidden_states, w_r)


def kernel(hidden_states, W):
    rw_tc, idx_tc = _tc_router(hidden_states, W)
    if SC_TOKENS == 0:
        return (rw_tc, idx_tc)
    w1, w2, i1, i2 = _sc_router(hidden_states, W)
    rw_sc = jnp.stack([w1[:, 0], w2[:, 0]], axis=-1)
    idx_sc = jnp.stack([i1[:, 0], i2[:, 0]], axis=-1)
    rw = jnp.concatenate([rw_tc, rw_sc], axis=0)
    idx = jnp.concatenate([idx_tc, idx_sc], axis=0)
    return (rw, idx)


# speedup vs baseline: 2.0265x; 1.1676x over previous
import jax
import jax.numpy as jnp
from jax import lax
from jax.experimental import pallas as pl
from jax.experimental.pallas import tpu as pltpu

D_MODEL = 2048
NUM_EXPERTS = 16
TOP_K = 2
TOKENS = 16384

BLOCK_T = 2048


def _router_body(hs_ref, w_ref, rw_ref, idx_ref):
    logits = lax.dot_general(
        hs_ref[...], w_ref[...],
        dimension_numbers=(((1,), (1,)), ((), ())),
        preferred_element_type=jnp.float32,
    )

    iota = lax.broadcasted_iota(jnp.int32, logits.shape, 1)
    m1 = jnp.max(logits, axis=-1, keepdims=True)
    i1 = jnp.min(jnp.where(logits == m1, iota, NUM_EXPERTS),
                 axis=-1, keepdims=True)
    masked = jnp.where(iota == i1, -jnp.inf, logits)
    m2 = jnp.max(masked, axis=-1, keepdims=True)
    i2 = jnp.min(jnp.where(masked == m2, iota, NUM_EXPERTS),
                 axis=-1, keepdims=True)

    e2 = jnp.exp(m2 - m1)
    denom = 1.0 + e2
    rw_ref[...] = jnp.concatenate([1.0 / denom, e2 / denom], axis=-1)
    idx_ref[...] = jnp.concatenate([i1, i2], axis=-1)


def kernel(hidden_states, W):
    n_blocks = TOKENS // BLOCK_T
    rw, idx = pl.pallas_call(
        _router_body,
        grid=(n_blocks,),
        in_specs=[
            pl.BlockSpec((BLOCK_T, D_MODEL), lambda i: (i, 0)),
            pl.BlockSpec((NUM_EXPERTS, D_MODEL), lambda i: (0, 0)),
        ],
        out_specs=[
            pl.BlockSpec((BLOCK_T, TOP_K), lambda i: (i, 0)),
            pl.BlockSpec((BLOCK_T, TOP_K), lambda i: (i, 0)),
        ],
        out_shape=[
            jax.ShapeDtypeStruct((TOKENS, TOP_K), jnp.float32),
            jax.ShapeDtypeStruct((TOKENS, TOP_K), jnp.int32),
        ],
        compiler_params=pltpu.CompilerParams(
            dimension_semantics=("parallel",),
        ),
    )(hidden_states, W)
    return (rw, idx)
